# Initial kernel scaffold; baseline (speedup 1.0000x reference)
import jax
import jax.numpy as jnp
from jax.experimental import pallas as pl


def kernel(x, edge_index, w1_0, b1_0, w2_0, b2_0, w1_1, b1_1, w2_1, b2_1, w1_2, b1_2, w2_2, b2_2, w1_3, b1_3, w2_3, b2_3):
    raise NotImplementedError("write your pallas kernel here")


# speedup vs baseline: 6.2904x; 6.2904x over previous
import functools

import jax
import jax.numpy as jnp
from jax import lax
from jax.experimental import pallas as pl
from jax.experimental.pallas import tpu as pltpu
from jax.experimental.pallas import tpu_sc as plsc

_N = 10000
_E = 320000
_D = 128
_NC = 2
_NS = 16
_NW = _NC * _NS
_EW = _E // _NW
_K = 80
_CH = _EW // _K
_RPT = 624
_RTAIL = _N - _NS * _RPT


def _sc_body(h_hbm, src_hbm, dst_hbm, out_hbm, acc, src_v, dst_v, rows_v, sem):
    cid = lax.axis_index("c")
    sid = lax.axis_index("s")
    g = cid * _NS + sid

    r0 = sid * _RPT
    pltpu.sync_copy(h_hbm.at[pl.ds(r0, _RPT)], acc.at[pl.ds(r0, _RPT)])

    @pl.when(sid == _NS - 1)
    def _():
        t0 = _NS * _RPT
        pltpu.sync_copy(h_hbm.at[pl.ds(t0, _RTAIL)], acc.at[pl.ds(t0, _RTAIL)])

    pltpu.sync_copy(src_hbm.at[g], src_v)
    pltpu.sync_copy(dst_hbm.at[g], dst_v)

    plsc.subcore_barrier()

    def chunk(i, carry):
        pltpu.async_copy(h_hbm.at[src_v.at[i]], rows_v, sem).wait()
        pltpu.sync_copy(rows_v, acc.at[dst_v.at[i]], add=True)
        return carry

    lax.fori_loop(0, _CH, chunk, 0)

    plsc.subcore_barrier()

    pltpu.sync_copy(acc.at[pl.ds(r0, _RPT)], out_hbm.at[cid, pl.ds(r0, _RPT)])

    @pl.when(sid == _NS - 1)
    def _():
        t0 = _NS * _RPT
        pltpu.sync_copy(acc.at[pl.ds(t0, _RTAIL)], out_hbm.at[cid, pl.ds(t0, _RTAIL)])


_sc_segsum = pl.kernel(
    _sc_body,
    out_type=jax.ShapeDtypeStruct((_NC, _N, _D), jnp.float32),
    mesh=plsc.VectorSubcoreMesh(core_axis_name="c", subcore_axis_name="s"),
    scratch_types=[
        pltpu.VMEM_SHARED((_N, _D), jnp.float32),
        pltpu.VMEM((_CH, _K), jnp.int32),
        pltpu.VMEM((_CH, _K), jnp.int32),
        pltpu.VMEM((_K, _D), jnp.float32),
        pltpu.SemaphoreType.DMA,
    ],
)


_BN = 2000


def _mlp_body(acc0, acc1, h, w1, b1, w2, b2, out, *, last):
    t = acc0[0] + acc1[0] - h[...]
    t = jnp.dot(t, w1[...], precision=lax.Precision.HIGHEST) + b1[...]
    t = jnp.maximum(t, 0.0)
    t = jnp.dot(t, w2[...], precision=lax.Precision.HIGHEST) + b2[...]
    if not last:
        t = jnp.maximum(t, 0.0)
    out[...] = t


def _mlp(acc, h, w1, b1, w2, b2, last):
    row = lambda i: (i, 0)
    full = lambda i: (0, 0)
    return pl.pallas_call(
        functools.partial(_mlp_body, last=last),
        grid=(_N // _BN,),
        in_specs=[
            pl.BlockSpec((1, _BN, _D), lambda i: (0, i, 0)),
            pl.BlockSpec((1, _BN, _D), lambda i: (1, i, 0)),
            pl.BlockSpec((_BN, _D), row),
            pl.BlockSpec((_D, _D), full),
            pl.BlockSpec((1, _D), full),
            pl.BlockSpec((_D, _D), full),
            pl.BlockSpec((1, _D), full),
        ],
        out_specs=pl.BlockSpec((_BN, _D), row),
        out_shape=jax.ShapeDtypeStruct((_N, _D), jnp.float32),
    )(acc, acc, h, w1, b1, w2, b2)


def kernel(x, edge_index, w1_0, b1_0, w2_0, b2_0, w1_1, b1_1, w2_1, b2_1,
           w1_2, b1_2, w2_2, b2_2, w1_3, b1_3, w2_3, b2_3):
    src = edge_index[0].reshape(_NW, _CH, _K)
    dst = edge_index[1].reshape(_NW, _CH, _K)
    params = [(w1_0, b1_0, w2_0, b2_0), (w1_1, b1_1, w2_1, b2_1),
              (w1_2, b1_2, w2_2, b2_2), (w1_3, b1_3, w2_3, b2_3)]
    h = x
    for l, (w1, b1, w2, b2) in enumerate(params):
        acc = _sc_segsum(h, src, dst)
        h = _mlp(acc, h, w1, b1.reshape(1, _D), w2, b2.reshape(1, _D),
                 last=(l == len(params) - 1))
    return h
